# baseline (speedup 1.0000x reference)
import functools

import jax
import jax.numpy as jnp
from jax import lax
from jax.experimental import pallas as pl
from jax.experimental.pallas import tpu as pltpu
from jax.experimental.pallas import tpu_sc as plsc


def _sc_corr_body(rows_per, C, x_hbm, t_hbm, out_hbm, xv, tv, accv):
    nc = plsc.get_sparse_core_info().num_cores
    wid = lax.axis_index("s") * nc + lax.axis_index("c")
    base = wid * rows_per * C
    pltpu.sync_copy(x_hbm.at[pl.ds(base, rows_per * C)], xv)
    pltpu.sync_copy(t_hbm.at[pl.ds(base, rows_per * C)], tv)

    lane_off = lax.iota(jnp.int32, 16) * C

    def group(g, acc):
        flat0 = g * (16 * C) + lane_off
        a = jnp.zeros((16,), jnp.float32)
        p = jnp.zeros((16,), jnp.float32)
        no = jnp.zeros((16,), jnp.float32)
        for c in range(C):
            idx = flat0 + c
            x = plsc.load_gather(xv, [idx])
            t = plsc.load_gather(tv, [idx])
            y = t.astype(jnp.float32)
            u = jnp.exp(-x)
            s = 1.0 / (1.0 + u)
            es = jnp.exp(s)
            a = a + (1.0 - y) * es
            p = p + y / es
            no = no + y
        nz = float(C) - no
        den = no * nz
        per = jnp.where(den > 0.0, (a * p) / jnp.maximum(den, 1.0), 0.0)
        return acc + per

    acc = lax.fori_loop(0, rows_per // 16, group, jnp.zeros((16,), jnp.float32))
    accv[...] = acc
    pltpu.sync_copy(accv, out_hbm.at[wid])


def _sc_corr_partials(x_flat, t_flat, B, C):
    info = plsc.get_sparse_core_info()
    nw = info.num_cores * info.num_subcores
    rows_per = B // nw
    mesh = plsc.VectorSubcoreMesh(core_axis_name="c", subcore_axis_name="s")
    k = functools.partial(
        pl.kernel,
        mesh=mesh,
        compiler_params=pltpu.CompilerParams(needs_layout_passes=False),
        out_type=jax.ShapeDtypeStruct((nw, 16), jnp.float32),
        scratch_types=[
            pltpu.VMEM((rows_per * C,), jnp.float32),
            pltpu.VMEM((rows_per * C,), jnp.int32),
            pltpu.VMEM((16,), jnp.float32),
        ],
    )(functools.partial(_sc_corr_body, rows_per, C))
    return k(x_flat, t_flat)


def _tc_bce_body(x_ref, t_ref, o_ref):
    x = x_ref[:]
    y = t_ref[:].astype(jnp.float32)
    bce = jnp.sum(jnp.maximum(x, 0.0) - x * y
                  + jnp.log1p(jnp.exp(-jnp.abs(x))))
    i = pl.program_id(0)

    @pl.when(i == 0)
    def _():
        o_ref[:] = jnp.zeros_like(o_ref)

    col = lax.broadcasted_iota(jnp.int32, (1, 128), 1)
    o_ref[:] += jnp.where(col == 0, bce, 0.0)


def _tc_bce_sum(x2d, t2d, grid=8):
    R, L = x2d.shape
    blk = R // grid
    out = pl.pallas_call(
        _tc_bce_body,
        grid=(grid,),
        in_specs=[
            pl.BlockSpec((blk, L), lambda i: (i, 0)),
            pl.BlockSpec((blk, L), lambda i: (i, 0)),
        ],
        out_specs=pl.BlockSpec((1, 128), lambda i: (0, 0)),
        out_shape=jax.ShapeDtypeStruct((1, 128), jnp.float32),
    )(x2d, t2d)
    return out[0, 0]


def kernel(logits, targets):
    B, C = logits.shape
    x_flat = logits.reshape(-1)
    t_flat = targets.reshape(-1)
    corr_partials = _sc_corr_partials(x_flat, t_flat, B, C)
    n = B * C
    lanes = 128
    rows = n // lanes
    bce_sum = _tc_bce_sum(x_flat.reshape(rows, lanes),
                          t_flat.reshape(rows, lanes))
    corr_mean = jnp.sum(corr_partials) / B
    bce_mean = bce_sum / n
    return 0.8 * bce_mean + 0.2 * corr_mean


# speedup vs baseline: 2.6542x; 2.6542x over previous
import jax
import jax.numpy as jnp
from jax import lax
from jax.experimental import pallas as pl


def _loss_body(x_ref, t_ref, o_ref):
    x = x_ref[:]
    y = t_ref[:].astype(jnp.float32)
    C = x.shape[1]

    u = jnp.exp(-x)
    w = 1.0 + u
    bce = jnp.sum(x * (1.0 - y) + jnp.log(w))
    s = 1.0 / w
    es = jnp.exp(s)
    a = jnp.sum(jnp.where(y == 0.0, es, 0.0), axis=1)
    p = jnp.sum(jnp.where(y == 0.0, 0.0, 1.0 / es), axis=1)
    n_o = jnp.sum(y, axis=1)
    n_z = C - n_o
    den = n_o * n_z
    per = jnp.where(den > 0.0, (a * p) / jnp.maximum(den, 1.0), 0.0)
    corr = jnp.sum(per)

    i = pl.program_id(0)

    @pl.when(i == 0)
    def _():
        o_ref[:] = jnp.zeros_like(o_ref)

    col = lax.broadcasted_iota(jnp.int32, (1, 128), 1)
    o_ref[:] += jnp.where(col == 0, bce, 0.0) + jnp.where(col == 1, corr, 0.0)


def kernel(logits, targets, grid=8):
    B, C = logits.shape
    blk = B // grid
    out = pl.pallas_call(
        _loss_body,
        grid=(grid,),
        in_specs=[
            pl.BlockSpec((blk, C), lambda i: (i, 0)),
            pl.BlockSpec((blk, C), lambda i: (i, 0)),
        ],
        out_specs=pl.BlockSpec((1, 128), lambda i: (0, 0)),
        out_shape=jax.ShapeDtypeStruct((1, 128), jnp.float32),
    )(logits, targets)
    bce_mean = out[0, 0] / (B * C)
    corr_mean = out[0, 1] / B
    return 0.8 * bce_mean + 0.2 * corr_mean
